# baseline (speedup 1.0000x reference)
import functools

import jax
import jax.numpy as jnp
from jax import lax
from jax.experimental import pallas as pl
from jax.experimental.pallas import tpu as pltpu
from jax.experimental.pallas import tpu_sc as plsc

B, S, D = 1024, 200, 64
N = B * S
NC, NS = 2, 16
NW = NC * NS
NBB = 8
NSQ = 4
BB = B // NBB
SQ = S // NSQ
V = 1000000
VP = V // 2
TCOLS = V // 128
TMAIN = TCOLS * 128
L = 16
NG = 4


def _pack_body(wt_hbm, tail_hbm, out2_hbm, b0, b1, p0, p1, tv,
               si0, si1, so0, so1):
    cid = lax.axis_index("c")
    sid = lax.axis_index("s")
    wid = sid * NC + cid
    ibufs, isems = (b0, b1), (si0, si1)
    pbufs, psems = (p0, p1), (so0, so1)
    iota = lax.iota(jnp.int32, L)
    rowv = [iota + L * j for j in range(4)]

    def tcol(k):
        return wid + NW * k

    def start_in(k, kk):
        pltpu.async_copy(wt_hbm.at[:, pl.ds(tcol(k) * 128, 128)],
                         ibufs[kk], isems[kk])

    def wait_in(kk):
        pltpu.make_async_copy(wt_hbm.at[:, pl.ds(0, 128)], ibufs[kk],
                              isems[kk]).wait()

    def out_slice(k):
        off = pl.multiple_of(tcol(k) * 64, 64)
        return out2_hbm.at[pl.ds(off, 64)]

    def slot(k, kk):
        @pl.when(tcol(k) < TCOLS)
        def _():
            wait_in(kk)
            blk, pb = ibufs[kk], pbufs[kk]

            @pl.when(k >= 2)
            def _():
                pltpu.make_async_copy(pb, out_slice(k), psems[kk]).wait()

            def prow(p, carry):
                for h in range(2):
                    cv = jnp.full((L,), 2 * p + h, jnp.int32)
                    for j in range(4):
                        val = plsc.load_gather(blk, [rowv[j], cv])
                        pb[p, pl.ds(h * D + j * L, L)] = val
                return carry

            lax.fori_loop(0, 64, prow, None, unroll=4)

            @pl.when(tcol(k + 2) < TCOLS)
            def _():
                start_in(k + 2, kk)

            pltpu.async_copy(pb, out_slice(k), psems[kk])

    start_in(0, 0)

    @pl.when(tcol(1) < TCOLS)
    def _():
        start_in(1, 1)

    def step(t, carry):
        slot(2 * t, 0)
        slot(2 * t + 1, 1)
        return carry

    nk = TCOLS // NW + 1
    lax.fori_loop(0, (nk + 1) // 2, step, None)

    @pl.when(tcol(nk - 2) < TCOLS)
    def _():
        kk = (nk - 2) % 2
        pltpu.make_async_copy(pbufs[kk], out_slice(nk - 2), psems[kk]).wait()

    @pl.when(tcol(nk - 1) < TCOLS)
    def _():
        kk = (nk - 1) % 2
        pltpu.make_async_copy(pbufs[kk], out_slice(nk - 1), psems[kk]).wait()

    @pl.when(wid == NW - 1)
    def _():
        pltpu.sync_copy(tail_hbm, tv)
        pltpu.sync_copy(tv, out2_hbm.at[pl.ds(TMAIN // 2, 32)])


def _emb_body(idxh_hbm, par_hbm, pos4_hbm, words2_hbm, out3_hbm,
              idxh_v, par_v, pos_v, g0, g1, g2, g3, p0, p1,
              sg0, sg1, sg2, sg3, so0, so1):
    gbufs = (g0, g1, g2, g3)
    gsems = (sg0, sg1, sg2, sg3)
    pbufs, psems = (p0, p1), (so0, so1)
    cid = lax.axis_index("c")
    sid = lax.axis_index("s")
    wid = sid * NC + cid
    bb = wid % NBB
    sq = wid // NBB
    pltpu.sync_copy(idxh_hbm.at[wid], idxh_v)
    pltpu.sync_copy(par_hbm.at[wid], par_v)
    pltpu.sync_copy(pos4_hbm.at[sq], pos_v)

    def start_gather(c, k):
        pltpu.async_copy(words2_hbm.at[idxh_v.at[c]], gbufs[k], gsems[k])

    def wait_gather(k):
        pltpu.make_async_copy(words2_hbm.at[idxh_v.at[0]], gbufs[k],
                              gsems[k]).wait()

    def out_slice(c):
        return out3_hbm.at[sq * SQ + c, :, pl.ds(bb * BB, BB)]

    iota = lax.iota(jnp.int32, L)
    rowv = [iota + g * L for g in range(BB // L)]

    def consume(c, k, j):
        wait_gather(k)
        buf, p_v = gbufs[k], pbufs[j]
        colb = [par_v[c, pl.ds(g * L, L)] for g in range(BB // L)]

        @pl.when(c >= 2)
        def _():
            pltpu.make_async_copy(p_v, out_slice(c), psems[j]).wait()

        def feat(cf, carry):
            cfv = jnp.full((L,), cf, jnp.int32)
            posb = plsc.load_gather(
                pos_v, [jnp.full((L,), c, jnp.int32), cfv])
            for g in range(BB // L):
                val = plsc.load_gather(buf, [rowv[g], colb[g] + cf])
                p_v[cf, pl.ds(g * L, L)] = val + posb
            return carry

        lax.fori_loop(0, D, feat, None, unroll=4)

        @pl.when(c + NG < SQ)
        def _():
            start_gather(c + NG, k)

        pltpu.async_copy(p_v, out_slice(c), psems[j])

    for k in range(NG):
        start_gather(k, k)

    def step(t, carry):
        c0 = NG * t
        for k in range(NG):
            consume(c0 + k, k, k % 2)
        return carry

    lax.fori_loop(0, SQ // NG, step, None)
    consume(jnp.int32(SQ - 2), 0, 0)
    consume(jnp.int32(SQ - 1), 1, 1)
    pltpu.make_async_copy(pbufs[0], out_slice(SQ - 2), psems[0]).wait()
    pltpu.make_async_copy(pbufs[1], out_slice(SQ - 1), psems[1]).wait()


def kernel(input_ids, words, pos_table):
    mesh = plsc.VectorSubcoreMesh(core_axis_name="c", subcore_axis_name="s")
    cp = pltpu.CompilerParams(use_tc_tiling_on_sc=True,
                              needs_layout_passes=False)

    words_t = words.T
    tail = words[TMAIN:].reshape(32, 2 * D)
    words2 = pl.kernel(
        _pack_body,
        out_type=jax.ShapeDtypeStruct((VP, 2 * D), jnp.float32),
        mesh=mesh,
        scratch_types=(
            [pltpu.VMEM((D, 2 * D), jnp.float32) for _ in range(2)]
            + [pltpu.VMEM((D, 2 * D), jnp.float32) for _ in range(2)]
            + [pltpu.VMEM((32, 2 * D), jnp.float32)]
            + [pltpu.SemaphoreType.DMA for _ in range(4)]
        ),
        compiler_params=cp,
    )(words_t, tail)

    idx_r = (input_ids.astype(jnp.int32).T
             .reshape(NSQ, SQ, NBB, BB).transpose(0, 2, 1, 3)
             .reshape(NW, SQ, BB))
    idx_half = idx_r >> 1
    idx_par = (idx_r & 1) << 6
    pos4 = pos_table[:S].reshape(NSQ, SQ, D)
    out3 = pl.kernel(
        _emb_body,
        out_type=jax.ShapeDtypeStruct((S, D, B), jnp.float32),
        mesh=mesh,
        scratch_types=(
            [pltpu.VMEM((SQ, BB), jnp.int32),
             pltpu.VMEM((SQ, BB), jnp.int32),
             pltpu.VMEM((SQ, D), jnp.float32)]
            + [pltpu.VMEM((BB, 2 * D), jnp.float32) for _ in range(NG)]
            + [pltpu.VMEM((D, BB), jnp.float32) for _ in range(2)]
            + [pltpu.SemaphoreType.DMA for _ in range(NG + 2)]
        ),
        compiler_params=cp,
    )(idx_half, idx_par, pos4, words2)
    return jnp.transpose(out3, (2, 0, 1))


# speedup vs baseline: 2.1124x; 2.1124x over previous
import functools

import jax
import jax.numpy as jnp
from jax import lax
from jax.experimental import pallas as pl
from jax.experimental.pallas import tpu as pltpu
from jax.experimental.pallas import tpu_sc as plsc

B, S, D = 1024, 200, 64
N = B * S
NC, NS = 2, 16
NW = NC * NS
PER_W = N // NW
CH = 128
NCH = PER_W // CH
POS2 = 2 * S
NG = 8
NO = 2
UNROLL = 8


def _emb_body(idx_hbm, pos2_hbm, words_hbm, out_hbm,
              idx_v, pos_v, g0, g1, g2, g3, g4, g5, g6, g7, o0, o1,
              sg0, sg1, sg2, sg3, sg4, sg5, sg6, sg7, so0, so1):
    gbufs = (g0, g1, g2, g3, g4, g5, g6, g7)
    gsems = (sg0, sg1, sg2, sg3, sg4, sg5, sg6, sg7)
    obufs = (o0, o1)
    osems = (so0, so1)
    cid = lax.axis_index("c")
    sid = lax.axis_index("s")
    wid = sid * NC + cid
    base = wid * PER_W
    pltpu.sync_copy(idx_hbm.at[pl.ds(base, PER_W)], idx_v)
    pltpu.sync_copy(pos2_hbm, pos_v)

    def start_gather(c, k):
        pltpu.async_copy(words_hbm.at[idx_v.at[pl.ds(c * CH, CH)]],
                         gbufs[k], gsems[k])

    def wait_gather(k):
        pltpu.make_async_copy(words_hbm.at[idx_v.at[pl.ds(0, CH)]],
                              gbufs[k], gsems[k]).wait()

    def out_slice(c):
        return out_hbm.at[pl.ds(base + c * CH, CH)]

    def consume(c, k, j):
        wait_gather(k)
        o_v = obufs[j]

        @pl.when(c >= NO)
        def _():
            pltpu.make_async_copy(o_v, out_slice(c), osems[j]).wait()

        rbase = lax.rem(c * CH, S)

        def add_row(r, carry):
            prow = rbase + r
            for jj in range(4):
                sl = pl.ds(jj * 16, 16)
                o_v[r, sl] = gbufs[k][r, sl] + pos_v[prow, sl]
            return carry

        lax.fori_loop(0, CH, add_row, None, unroll=UNROLL)

        @pl.when(c + NG < NCH)
        def _():
            start_gather(c + NG, k)

        pltpu.async_copy(o_v, out_slice(c), osems[j])

    for k in range(NG):
        start_gather(k, k)

    def step(t, carry):
        c0 = NG * t
        for k in range(NG):
            consume(c0 + k, k, k % NO)
        return carry

    lax.fori_loop(0, NCH // NG, step, None)
    consume(jnp.int32(NCH - 2), 0, 0)
    consume(jnp.int32(NCH - 1), 1, 1)
    pltpu.make_async_copy(obufs[0], out_slice(NCH - 2), osems[0]).wait()
    pltpu.make_async_copy(obufs[1], out_slice(NCH - 1), osems[1]).wait()


def kernel(input_ids, words, pos_table):
    idx = input_ids.reshape(-1).astype(jnp.int32)
    pos2 = jnp.concatenate([pos_table[:S], pos_table[:S]], axis=0)
    mesh = plsc.VectorSubcoreMesh(core_axis_name="c", subcore_axis_name="s")
    out = pl.kernel(
        _emb_body,
        out_type=jax.ShapeDtypeStruct((N, D), jnp.float32),
        mesh=mesh,
        scratch_types=(
            [pltpu.VMEM((PER_W,), jnp.int32),
             pltpu.VMEM((POS2, D), jnp.float32)]
            + [pltpu.VMEM((CH, D), jnp.float32) for _ in range(NG + NO)]
            + [pltpu.SemaphoreType.DMA for _ in range(NG + NO)]
        ),
        compiler_params=pltpu.CompilerParams(use_tc_tiling_on_sc=False),
    )(idx, pos2, words)
    return out.reshape(B, S, D)


# speedup vs baseline: 2.1214x; 1.0043x over previous
import functools

import jax
import jax.numpy as jnp
from jax import lax
from jax.experimental import pallas as pl
from jax.experimental.pallas import tpu as pltpu
from jax.experimental.pallas import tpu_sc as plsc

B, S, D = 1024, 200, 64
N = B * S
NC, NS = 2, 16
NW = NC * NS
PER_W = N // NW
CH = 128
NCH = PER_W // CH
POS2 = 2 * S
NG = 4
NO = 2
UNROLL = 8


def _emb_body(idx_hbm, pos2_hbm, words_hbm, out_hbm,
              idx_v, pos_v, g0, g1, g2, g3, o0, o1,
              sg0, sg1, sg2, sg3, so0, so1):
    gbufs = (g0, g1, g2, g3)
    gsems = (sg0, sg1, sg2, sg3)
    obufs = (o0, o1)
    osems = (so0, so1)
    cid = lax.axis_index("c")
    sid = lax.axis_index("s")
    wid = sid * NC + cid
    base = wid * PER_W
    pltpu.sync_copy(idx_hbm.at[pl.ds(base, PER_W)], idx_v)
    pltpu.sync_copy(pos2_hbm, pos_v)

    def start_gather(c, k):
        pltpu.async_copy(words_hbm.at[idx_v.at[pl.ds(c * CH, CH)]],
                         gbufs[k], gsems[k])

    def wait_gather(k):
        pltpu.make_async_copy(words_hbm.at[idx_v.at[pl.ds(0, CH)]],
                              gbufs[k], gsems[k]).wait()

    def out_slice(c):
        return out_hbm.at[pl.ds(base + c * CH, CH)]

    def consume(c, k, j):
        wait_gather(k)
        o_v = obufs[j]

        @pl.when(c >= NO)
        def _():
            pltpu.make_async_copy(o_v, out_slice(c), osems[j]).wait()

        rbase = lax.rem(c * CH, S)

        def add_row(r, carry):
            prow = rbase + r
            for jj in range(4):
                sl = pl.ds(jj * 16, 16)
                o_v[r, sl] = gbufs[k][r, sl] + pos_v[prow, sl]
            return carry

        lax.fori_loop(0, CH, add_row, None, unroll=UNROLL)

        @pl.when(c + NG < NCH)
        def _():
            start_gather(c + NG, k)

        pltpu.async_copy(o_v, out_slice(c), osems[j])

    for k in range(NG):
        start_gather(k, k)

    def step(t, carry):
        c0 = NG * t
        for k in range(NG):
            consume(c0 + k, k, k % NO)
        return carry

    lax.fori_loop(0, NCH // NG, step, None)
    consume(jnp.int32(NCH - 2), 0, 0)
    consume(jnp.int32(NCH - 1), 1, 1)
    pltpu.make_async_copy(obufs[0], out_slice(NCH - 2), osems[0]).wait()
    pltpu.make_async_copy(obufs[1], out_slice(NCH - 1), osems[1]).wait()


def kernel(input_ids, words, pos_table):
    idx = input_ids.reshape(-1).astype(jnp.int32)
    pos2 = jnp.concatenate([pos_table[:S], pos_table[:S]], axis=0)
    mesh = plsc.VectorSubcoreMesh(core_axis_name="c", subcore_axis_name="s")
    out = pl.kernel(
        _emb_body,
        out_type=jax.ShapeDtypeStruct((N, D), jnp.float32),
        mesh=mesh,
        scratch_types=(
            [pltpu.VMEM((PER_W,), jnp.int32),
             pltpu.VMEM((POS2, D), jnp.float32)]
            + [pltpu.VMEM((CH, D), jnp.float32) for _ in range(NG + NO)]
            + [pltpu.SemaphoreType.DMA for _ in range(NG + NO)]
        ),
        compiler_params=pltpu.CompilerParams(use_tc_tiling_on_sc=False),
    )(idx, pos2, words)
    return out.reshape(B, S, D)
